# baseline (speedup 1.0000x reference)
import jax
import jax.numpy as jnp
import numpy as np
from jax import lax
from jax.experimental import pallas as pl
from jax.experimental.pallas import tpu as pltpu

_K = 9
_NEG_MIN = np.int32(-2147483648)
_MASK31 = np.int32(0x7FFFFFFF)


def _gelu(v):
    return jax.nn.gelu(v)


def _mm(a, b):
    return lax.dot_general(a, b, (((1,), (0,)), ((), ())),
                           preferred_element_type=jnp.float32)


def _vig_body(x_ref, in1_W1, in1_b1, in1_W2, in1_b2, conv_Wh, conv_Wa, conv_b,
              out1_W1, out1_b1, out1_W2, out1_b2,
              in2_W1, in2_b1, in2_W2, in2_b2,
              out2_W1, out2_b1, out2_W2, out2_b2,
              out_ref, S_ref, h_ref, agg_ref):
    x = x_ref[0]
    N = x.shape[0]

    h = _mm(_gelu(_mm(x, in1_W1[...]) + in1_b1[...]), in1_W2[...]) + in1_b2[...]
    h_ref[...] = h.astype(jnp.bfloat16)

    sim = lax.dot_general(x, x, (((1,), (1,)), ((), ())),
                          preferred_element_type=jnp.float32)
    b = lax.bitcast_convert_type(sim, jnp.int32)
    S_ref[...] = b ^ (lax.shift_right_arithmetic(b, 31) & _MASK31)

    iota = lax.broadcasted_iota(jnp.int32, (N, N), 1)
    for t in range(_K):
        S = S_ref[...]
        m = jnp.max(S, axis=1, keepdims=True)
        idx = jnp.min(jnp.where(S == m, iota, np.int32(N)), axis=1,
                      keepdims=True)
        onehot = iota == idx
        sel = lax.dot_general(onehot.astype(jnp.bfloat16), h_ref[...],
                              (((1,), (0,)), ((), ())),
                              preferred_element_type=jnp.float32)
        if t == 0:
            agg_ref[...] = sel
        else:
            agg_ref[...] = jnp.maximum(agg_ref[...], sel)
        S_ref[...] = jnp.where(onehot, _NEG_MIN, S)

    agg = agg_ref[...] - h
    u = _mm(h, conv_Wh[...]) + _mm(agg, conv_Wa[...]) + conv_b[...]
    g = _gelu(u)
    h2 = _mm(_gelu(_mm(g, out1_W1[...]) + out1_b1[...]), out1_W2[...]) + out1_b2[...]
    hh = h2 + x

    t1 = _mm(_gelu(_mm(hh, in2_W1[...]) + in2_b1[...]), in2_W2[...]) + in2_b2[...]
    t2 = _gelu(t1)
    t3 = _mm(_gelu(_mm(t2, out2_W1[...]) + out2_b1[...]), out2_W2[...]) + out2_b2[...]
    out_ref[0] = t3 + hh


@jax.jit
def kernel(x, in1_W1, in1_b1, in1_W2, in1_b2, conv_W, conv_b,
           out1_W1, out1_b1, out1_W2, out1_b2,
           in2_W1, in2_b1, in2_W2, in2_b2,
           out2_W1, out2_b1, out2_W2, out2_b2):
    B, N, C = x.shape
    V = conv_W.T
    conv_Wh = V[0::2]
    conv_Wa = V[1::2]

    def row(v):
        return v.reshape(1, -1)

    full = lambda s: pl.BlockSpec(s, lambda b: (0,) * len(s))
    wspecs = [
        full((C, C)), full((1, C)), full((C, C)), full((1, C)),
        full((C, C)), full((C, C)), full((1, C)),
        full((C, C)), full((1, C)), full((C, C)), full((1, C)),
        full((C, 4 * C)), full((1, 4 * C)), full((4 * C, C)), full((1, C)),
        full((C, 4 * C)), full((1, 4 * C)), full((4 * C, C)), full((1, C)),
    ]
    return pl.pallas_call(
        _vig_body,
        grid=(B,),
        in_specs=[pl.BlockSpec((1, N, C), lambda b: (b, 0, 0))] + wspecs,
        out_specs=pl.BlockSpec((1, N, C), lambda b: (b, 0, 0)),
        out_shape=jax.ShapeDtypeStruct((B, N, C), jnp.float32),
        scratch_shapes=[
            pltpu.VMEM((N, N), jnp.int32),
            pltpu.VMEM((N, C), jnp.bfloat16),
            pltpu.VMEM((N, C), jnp.float32),
        ],
    )(x, in1_W1, row(in1_b1), in1_W2, row(in1_b2),
      conv_Wh, conv_Wa, row(conv_b),
      out1_W1, row(out1_b1), out1_W2, row(out1_b2),
      in2_W1, row(in2_b1), in2_W2, row(in2_b2),
      out2_W1, row(out2_b1), out2_W2, row(out2_b2))


# speedup vs baseline: 1.4812x; 1.4812x over previous
import jax
import jax.numpy as jnp
import numpy as np
from jax import lax
from jax.experimental import pallas as pl
from jax.experimental.pallas import tpu as pltpu

_K = 9
_NEG_MIN = np.int32(-2147483648)
_MASK31 = np.int32(0x7FFFFFFF)


def _gelu(v):
    return jax.nn.gelu(v)


def _mm(a, b):
    return lax.dot_general(a, b, (((1,), (0,)), ((), ())),
                           preferred_element_type=jnp.float32)


def _vig_body(x_ref, in1_W1, in1_b1, in1_W2, in1_b2, conv_Wh, conv_Wa, conv_b,
              out1_W1, out1_b1, out1_W2, out1_b2,
              in2_W1, in2_b1, in2_W2, in2_b2,
              out2_W1, out2_b1, out2_W2, out2_b2,
              out_ref, S_ref, h_ref, agg_ref):
    x = x_ref[0]
    N = x.shape[0]

    h = _mm(_gelu(_mm(x, in1_W1[...]) + in1_b1[...]), in1_W2[...]) + in1_b2[...]
    h_ref[...] = h.astype(jnp.bfloat16)

    sim = lax.dot_general(x, x, (((1,), (1,)), ((), ())),
                          preferred_element_type=jnp.float32)
    b = lax.bitcast_convert_type(sim, jnp.int32)
    s = b ^ (lax.shift_right_arithmetic(b, 31) & _MASK31)
    packed = (s & np.int32(-1024)) | (np.int32(1023) -
                                      lax.broadcasted_iota(jnp.int32, (N, N), 1))
    S_ref[...] = lax.bitcast_convert_type(
        packed ^ (lax.shift_right_arithmetic(packed, 31) & _MASK31),
        jnp.float32)

    iota = lax.broadcasted_iota(jnp.int32, (N, N), 1)
    fprev = None
    for t in range(_K):
        F = S_ref[...]
        masked = F if t == 0 else jnp.where(F < fprev, F, np.float32(-np.inf))
        m = jnp.max(masked, axis=1, keepdims=True)
        mp = lax.bitcast_convert_type(m, jnp.int32)
        mp = mp ^ (lax.shift_right_arithmetic(mp, 31) & _MASK31)
        idx = np.int32(1023) - (mp & np.int32(1023))
        onehot = iota == idx
        sel = lax.dot_general(onehot.astype(jnp.bfloat16), h_ref[...],
                              (((1,), (0,)), ((), ())),
                              preferred_element_type=jnp.float32)
        if t == 0:
            agg_ref[...] = sel
        else:
            agg_ref[...] = jnp.maximum(agg_ref[...], sel)
        fprev = m

    agg = agg_ref[...] - h
    u = _mm(h, conv_Wh[...]) + _mm(agg, conv_Wa[...]) + conv_b[...]
    g = _gelu(u)
    h2 = _mm(_gelu(_mm(g, out1_W1[...]) + out1_b1[...]), out1_W2[...]) + out1_b2[...]
    hh = h2 + x

    t1 = _mm(_gelu(_mm(hh, in2_W1[...]) + in2_b1[...]), in2_W2[...]) + in2_b2[...]
    t2 = _gelu(t1)
    t3 = _mm(_gelu(_mm(t2, out2_W1[...]) + out2_b1[...]), out2_W2[...]) + out2_b2[...]
    out_ref[0] = t3 + hh


@jax.jit
def kernel(x, in1_W1, in1_b1, in1_W2, in1_b2, conv_W, conv_b,
           out1_W1, out1_b1, out1_W2, out1_b2,
           in2_W1, in2_b1, in2_W2, in2_b2,
           out2_W1, out2_b1, out2_W2, out2_b2):
    B, N, C = x.shape
    V = conv_W.T
    conv_Wh = V[0::2]
    conv_Wa = V[1::2]

    def row(v):
        return v.reshape(1, -1)

    full = lambda s: pl.BlockSpec(s, lambda b: (0,) * len(s))
    wspecs = [
        full((C, C)), full((1, C)), full((C, C)), full((1, C)),
        full((C, C)), full((C, C)), full((1, C)),
        full((C, C)), full((1, C)), full((C, C)), full((1, C)),
        full((C, 4 * C)), full((1, 4 * C)), full((4 * C, C)), full((1, C)),
        full((C, 4 * C)), full((1, 4 * C)), full((4 * C, C)), full((1, C)),
    ]
    return pl.pallas_call(
        _vig_body,
        grid=(B,),
        in_specs=[pl.BlockSpec((1, N, C), lambda b: (b, 0, 0))] + wspecs,
        out_specs=pl.BlockSpec((1, N, C), lambda b: (b, 0, 0)),
        out_shape=jax.ShapeDtypeStruct((B, N, C), jnp.float32),
        scratch_shapes=[
            pltpu.VMEM((N, N), jnp.float32),
            pltpu.VMEM((N, C), jnp.bfloat16),
            pltpu.VMEM((N, C), jnp.float32),
        ],
    )(x, in1_W1, row(in1_b1), in1_W2, row(in1_b2),
      conv_Wh, conv_Wa, row(conv_b),
      out1_W1, row(out1_b1), out1_W2, row(out1_b2),
      in2_W1, row(in2_b1), in2_W2, row(in2_b2),
      out2_W1, row(out2_b1), out2_W2, row(out2_b2))
